# baseline (device time: 18202 ns/iter reference)
import functools

import jax
import jax.numpy as jnp
from jax import lax
from jax.experimental import pallas as pl
from jax.experimental.pallas import tpu as pltpu

N_DEV = 4


def kernel(A, B):
    m, k = A.shape
    _, n = B.shape
    m_out = m // N_DEV

    def body(a_ref, b_ref, out_ref, send_ref, recv_ref, scl_send_ref,
             scl_recv_ref, a16_ref, b16_ref, send_sems, recv_sems,
             scl_send_sems, scl_recv_sems):
        my = lax.axis_index("i")

        barrier_sem = pltpu.get_barrier_semaphore()
        for off in (1, 2, 3):
            pl.semaphore_signal(
                barrier_sem, inc=1,
                device_id=((my + off) % N_DEV,),
                device_id_type=pl.DeviceIdType.MESH,
            )
        pl.semaphore_wait(barrier_sem, 3)

        a16_ref[:, :] = a_ref[:, :].astype(jnp.bfloat16)
        b16_ref[:, :] = b_ref[:, :].astype(jnp.bfloat16)

        rdmas = []
        for off in (2, 1, 3):
            tgt = (my + off) % N_DEV
            slot = off - 1
            part = jnp.dot(
                a16_ref[pl.ds(tgt * m_out, m_out), :], b16_ref[:, :],
                preferred_element_type=jnp.float32,
            )
            absmax = jnp.maximum(jnp.max(jnp.abs(part)), 1e-30)
            send_ref[slot] = jnp.clip(
                jnp.round(part * (127.0 / absmax)), -127.0, 127.0
            ).astype(jnp.int8)
            scl_send_ref[slot] = jnp.full(
                (8, 128), absmax * (1.0 / 127.0), jnp.float32
            )
            scl_rdma = pltpu.make_async_remote_copy(
                src_ref=scl_send_ref.at[slot],
                dst_ref=scl_recv_ref.at[3 - off],
                send_sem=scl_send_sems.at[slot],
                recv_sem=scl_recv_sems.at[3 - off],
                device_id=(tgt,),
                device_id_type=pl.DeviceIdType.MESH,
            )
            scl_rdma.start()
            rdma = pltpu.make_async_remote_copy(
                src_ref=send_ref.at[slot],
                dst_ref=recv_ref.at[3 - off],
                send_sem=send_sems.at[slot],
                recv_sem=recv_sems.at[3 - off],
                device_id=(tgt,),
                device_id_type=pl.DeviceIdType.MESH,
            )
            rdma.start()
            rdmas.append((rdma, scl_rdma))

        own = jnp.dot(
            a16_ref[pl.ds(my * m_out, m_out), :], b16_ref[:, :],
            preferred_element_type=jnp.float32,
        )

        acc = own
        for i, rslot in enumerate((1, 2, 0)):
            rdma, scl_rdma = rdmas[i]
            scl_rdma.wait_recv()
            rdma.wait_recv()
            dq = recv_ref[rslot].astype(jnp.float32) * scl_recv_ref[rslot][0, 0]
            if i < 2:
                acc = acc + dq
            else:
                out_ref[:, :] = acc + dq

        for rdma, scl_rdma in rdmas:
            rdma.wait_send()
            scl_rdma.wait_send()

        @functools.partial(pl.run_scoped, sem=pltpu.SemaphoreType.REGULAR)
        def _(sem):
            for off in (1, 2, 3):
                pl.semaphore_signal(
                    sem, inc=1,
                    device_id=((my + off) % N_DEV,),
                    device_id_type=pl.DeviceIdType.MESH,
                )
            pl.semaphore_wait(sem, 3)

    return pl.pallas_call(
        body,
        out_shape=jax.ShapeDtypeStruct((m_out, n), jnp.float32),
        in_specs=[
            pl.BlockSpec(memory_space=pltpu.VMEM),
            pl.BlockSpec(memory_space=pltpu.VMEM),
        ],
        out_specs=pl.BlockSpec(memory_space=pltpu.VMEM),
        scratch_shapes=[
            pltpu.VMEM((N_DEV - 1, m_out, n), jnp.int8),
            pltpu.VMEM((N_DEV - 1, m_out, n), jnp.int8),
            pltpu.VMEM((N_DEV - 1, 8, 128), jnp.float32),
            pltpu.VMEM((N_DEV - 1, 8, 128), jnp.float32),
            pltpu.VMEM((m, k), jnp.bfloat16),
            pltpu.VMEM((k, n), jnp.bfloat16),
            pltpu.SemaphoreType.DMA((N_DEV - 1,)),
            pltpu.SemaphoreType.DMA((N_DEV - 1,)),
            pltpu.SemaphoreType.DMA((N_DEV - 1,)),
            pltpu.SemaphoreType.DMA((N_DEV - 1,)),
        ],
        compiler_params=pltpu.CompilerParams(collective_id=0),
    )(A, B)


# device time: 15864 ns/iter; 1.1474x vs baseline; 1.1474x over previous
import functools

import jax
import jax.numpy as jnp
from jax import lax
from jax.experimental import pallas as pl
from jax.experimental.pallas import tpu as pltpu

N_DEV = 4


def kernel(A, B):
    m, k = A.shape
    _, n = B.shape
    m_out = m // N_DEV

    def body(a_ref, b_ref, out_ref, send_ref, recv_ref, scl_send_ref,
             scl_recv_ref, a16_ref, b16_ref, send_sems, recv_sems,
             scl_send_sems, scl_recv_sems):
        my = lax.axis_index("i")

        barrier_sem = pltpu.get_barrier_semaphore()
        for off in (1, 2, 3):
            pl.semaphore_signal(
                barrier_sem, inc=1,
                device_id=((my + off) % N_DEV,),
                device_id_type=pl.DeviceIdType.MESH,
            )

        a16_ref[:, :] = a_ref[:, :].astype(jnp.bfloat16)
        b16_ref[:, :] = b_ref[:, :].astype(jnp.bfloat16)

        rdmas = []
        for off in (2, 1, 3):
            tgt = (my + off) % N_DEV
            slot = off - 1
            part = jnp.dot(
                a16_ref[pl.ds(tgt * m_out, m_out), :], b16_ref[:, :],
                preferred_element_type=jnp.float32,
            )
            absmax = jnp.maximum(jnp.max(jnp.abs(part)), 1e-30)
            send_ref[slot] = jnp.clip(
                jnp.round(part * (127.0 / absmax)), -127.0, 127.0
            ).astype(jnp.int8)
            scl_send_ref[slot] = jnp.full(
                (8, 128), absmax * (1.0 / 127.0), jnp.float32
            )
            if off == 2:
                pl.semaphore_wait(barrier_sem, 3)
            scl_rdma = pltpu.make_async_remote_copy(
                src_ref=scl_send_ref.at[slot],
                dst_ref=scl_recv_ref.at[3 - off],
                send_sem=scl_send_sems.at[slot],
                recv_sem=scl_recv_sems.at[3 - off],
                device_id=(tgt,),
                device_id_type=pl.DeviceIdType.MESH,
            )
            scl_rdma.start()
            rdma = pltpu.make_async_remote_copy(
                src_ref=send_ref.at[slot],
                dst_ref=recv_ref.at[3 - off],
                send_sem=send_sems.at[slot],
                recv_sem=recv_sems.at[3 - off],
                device_id=(tgt,),
                device_id_type=pl.DeviceIdType.MESH,
            )
            rdma.start()
            rdmas.append((rdma, scl_rdma))

        own = jnp.dot(
            a16_ref[pl.ds(my * m_out, m_out), :], b16_ref[:, :],
            preferred_element_type=jnp.float32,
        )

        acc = own
        for i, rslot in enumerate((1, 2, 0)):
            rdma, scl_rdma = rdmas[i]
            scl_rdma.wait_recv()
            rdma.wait_recv()
            dq = recv_ref[rslot].astype(jnp.float32) * scl_recv_ref[rslot][0, 0]
            if i < 2:
                acc = acc + dq
            else:
                out_ref[:, :] = acc + dq

        for rdma, scl_rdma in rdmas:
            rdma.wait_send()
            scl_rdma.wait_send()


    return pl.pallas_call(
        body,
        out_shape=jax.ShapeDtypeStruct((m_out, n), jnp.float32),
        in_specs=[
            pl.BlockSpec(memory_space=pltpu.VMEM),
            pl.BlockSpec(memory_space=pltpu.VMEM),
        ],
        out_specs=pl.BlockSpec(memory_space=pltpu.VMEM),
        scratch_shapes=[
            pltpu.VMEM((N_DEV - 1, m_out, n), jnp.int8),
            pltpu.VMEM((N_DEV - 1, m_out, n), jnp.int8),
            pltpu.VMEM((N_DEV - 1, 8, 128), jnp.float32),
            pltpu.VMEM((N_DEV - 1, 8, 128), jnp.float32),
            pltpu.VMEM((m, k), jnp.bfloat16),
            pltpu.VMEM((k, n), jnp.bfloat16),
            pltpu.SemaphoreType.DMA((N_DEV - 1,)),
            pltpu.SemaphoreType.DMA((N_DEV - 1,)),
            pltpu.SemaphoreType.DMA((N_DEV - 1,)),
            pltpu.SemaphoreType.DMA((N_DEV - 1,)),
        ],
        compiler_params=pltpu.CompilerParams(collective_id=0),
    )(A, B)


# device time: 15618 ns/iter; 1.1655x vs baseline; 1.0158x over previous
import jax
import jax.numpy as jnp
from jax import lax
from jax.experimental import pallas as pl
from jax.experimental.pallas import tpu as pltpu

N_DEV = 4
PIECES = 2


def kernel(A, B):
    m, k = A.shape
    _, n = B.shape
    m_out = m // N_DEV
    m_pc = m_out // PIECES

    def body(a_ref, b_ref, out_ref, send_ref, recv_ref, scl_send_ref,
             scl_recv_ref, a16_ref, b16_ref, send_sems, recv_sems,
             scl_send_sems, scl_recv_sems):
        my = lax.axis_index("i")

        barrier_sem = pltpu.get_barrier_semaphore()
        for off in (1, 2, 3):
            pl.semaphore_signal(
                barrier_sem, inc=1,
                device_id=((my + off) % N_DEV,),
                device_id_type=pl.DeviceIdType.MESH,
            )

        a16_ref[:, :] = a_ref[:, :].astype(jnp.bfloat16)
        b16_ref[:, :] = b_ref[:, :].astype(jnp.bfloat16)

        rdmas = []
        first = True
        for off in (2, 1, 3):
            tgt = (my + off) % N_DEV
            slot = off - 1
            for half in range(PIECES):
                p = slot * PIECES + half
                rp = (3 - off) * PIECES + half
                part = jnp.dot(
                    a16_ref[pl.ds(tgt * m_out + half * m_pc, m_pc), :],
                    b16_ref[:, :],
                    preferred_element_type=jnp.float32,
                )
                absmax = jnp.maximum(jnp.max(jnp.abs(part)), 1e-30)
                send_ref[p] = jnp.clip(
                    jnp.round(part * (127.0 / absmax)), -127.0, 127.0
                ).astype(jnp.int8)
                scl_send_ref[p] = jnp.full(
                    (8, 128), absmax * (1.0 / 127.0), jnp.float32
                )
                if first:
                    pl.semaphore_wait(barrier_sem, 3)
                    first = False
                scl_rdma = pltpu.make_async_remote_copy(
                    src_ref=scl_send_ref.at[p],
                    dst_ref=scl_recv_ref.at[rp],
                    send_sem=scl_send_sems.at[p],
                    recv_sem=scl_recv_sems.at[rp],
                    device_id=(tgt,),
                    device_id_type=pl.DeviceIdType.MESH,
                )
                scl_rdma.start()
                rdma = pltpu.make_async_remote_copy(
                    src_ref=send_ref.at[p],
                    dst_ref=recv_ref.at[rp],
                    send_sem=send_sems.at[p],
                    recv_sem=recv_sems.at[rp],
                    device_id=(tgt,),
                    device_id_type=pl.DeviceIdType.MESH,
                )
                rdma.start()
                rdmas.append((rdma, scl_rdma, rp, half))

        out_ref[:, :] = jnp.dot(
            a16_ref[pl.ds(my * m_out, m_out), :], b16_ref[:, :],
            preferred_element_type=jnp.float32,
        )

        for rdma, scl_rdma, rp, half in rdmas:
            scl_rdma.wait_recv()
            rdma.wait_recv()
            rows = pl.ds(half * m_pc, m_pc)
            out_ref[rows, :] = out_ref[rows, :] + (
                recv_ref[rp].astype(jnp.float32) * scl_recv_ref[rp][0, 0]
            )

        for rdma, scl_rdma, _, _ in rdmas:
            rdma.wait_send()
            scl_rdma.wait_send()


    n_pc = (N_DEV - 1) * PIECES
    return pl.pallas_call(
        body,
        out_shape=jax.ShapeDtypeStruct((m_out, n), jnp.float32),
        in_specs=[
            pl.BlockSpec(memory_space=pltpu.VMEM),
            pl.BlockSpec(memory_space=pltpu.VMEM),
        ],
        out_specs=pl.BlockSpec(memory_space=pltpu.VMEM),
        scratch_shapes=[
            pltpu.VMEM((n_pc, m_pc, n), jnp.int8),
            pltpu.VMEM((n_pc, m_pc, n), jnp.int8),
            pltpu.VMEM((n_pc, 8, 128), jnp.float32),
            pltpu.VMEM((n_pc, 8, 128), jnp.float32),
            pltpu.VMEM((m, k), jnp.bfloat16),
            pltpu.VMEM((k, n), jnp.bfloat16),
            pltpu.SemaphoreType.DMA((n_pc,)),
            pltpu.SemaphoreType.DMA((n_pc,)),
            pltpu.SemaphoreType.DMA((n_pc,)),
            pltpu.SemaphoreType.DMA((n_pc,)),
        ],
        compiler_params=pltpu.CompilerParams(collective_id=0),
    )(A, B)
